# baseline (speedup 1.0000x reference)
import functools

import jax
import jax.numpy as jnp
from jax.experimental import pallas as pl
from jax.experimental.pallas import tpu as pltpu


def _fused_kernel(item_id_ref, x_ref, w_ih_ref, w_hh_ref, b_ih_ref,
                  b_hh_ref, item_ref, user_ref, w_out_ref, b_out_ref,
                  out_ref, *, seq_len):
    del item_id_ref
    x = x_ref[...].reshape(x_ref.shape[0], x_ref.shape[2])
    xb = x.astype(jnp.bfloat16)
    xr = (jnp.dot(xb, w_ih_ref[0].astype(jnp.bfloat16),
                  preferred_element_type=jnp.float32)
          + (b_ih_ref[0] + b_hh_ref[0]))
    xz = (jnp.dot(xb, w_ih_ref[1].astype(jnp.bfloat16),
                  preferred_element_type=jnp.float32)
          + (b_ih_ref[1] + b_hh_ref[1]))
    xn = (jnp.dot(xb, w_ih_ref[2].astype(jnp.bfloat16),
                  preferred_element_type=jnp.float32)
          + b_ih_ref[2])

    ur = w_hh_ref[0].astype(jnp.bfloat16)
    uz = w_hh_ref[1].astype(jnp.bfloat16)
    un = w_hh_ref[2].astype(jnp.bfloat16)
    b_hn = b_hh_ref[2]

    H = ur.shape[0]
    h = jnp.zeros((1, H), jnp.float32)
    h_sum = jnp.zeros((1, H), jnp.float32)

    for t in range(seq_len):
        hb = h.astype(jnp.bfloat16)
        hr = jnp.dot(hb, ur, preferred_element_type=jnp.float32)
        hz = jnp.dot(hb, uz, preferred_element_type=jnp.float32)
        hn = jnp.dot(hb, un, preferred_element_type=jnp.float32)
        r = jax.nn.sigmoid(xr[t:t + 1, :] + hr)
        z = jax.nn.sigmoid(xz[t:t + 1, :] + hz)
        n = jnp.tanh(xn[t:t + 1, :] + r * (hn + b_hn))
        h = n + z * (h - n)
        h_sum = h_sum + h

    mean_h = h_sum * (1.0 / float(seq_len))

    scale = item_ref[0] * mean_h
    mul = (user_ref[...] * scale).astype(jnp.bfloat16)
    logits = (jnp.dot(mul, w_out_ref[...].astype(jnp.bfloat16),
                      preferred_element_type=jnp.float32)
              + b_out_ref[...])
    m = jnp.max(logits, axis=-1, keepdims=True)
    e = jnp.exp(logits - m)
    out_ref[...] = e / jnp.sum(e, axis=-1, keepdims=True)


def kernel(item_table, user_table, w_ih, w_hh, b_ih, b_hh, w_out, b_out,
           item_id, user_ids, word_embeddings):
    seq_len, batch, emb_dim = word_embeddings.shape
    hidden = w_hh.shape[-1]
    rating_range = w_out.shape[-1]
    assert batch == 1 and hidden == emb_dim

    user_emb = user_table[jnp.asarray(user_ids)]
    num_users = user_emb.shape[0]
    item_idx = jnp.reshape(item_id, (1,))

    kern = functools.partial(_fused_kernel, seq_len=seq_len)
    grid_spec = pltpu.PrefetchScalarGridSpec(
        num_scalar_prefetch=1,
        grid=(1,),
        in_specs=[
            pl.BlockSpec((seq_len, 1, emb_dim), lambda i, idx: (0, 0, 0)),
            pl.BlockSpec((3, emb_dim, hidden), lambda i, idx: (0, 0, 0)),
            pl.BlockSpec((3, hidden, hidden), lambda i, idx: (0, 0, 0)),
            pl.BlockSpec((3, 1, hidden), lambda i, idx: (0, 0, 0)),
            pl.BlockSpec((3, 1, hidden), lambda i, idx: (0, 0, 0)),
            pl.BlockSpec((1, 1, emb_dim), lambda i, idx: (idx[0], 0, 0)),
            pl.BlockSpec((num_users, emb_dim), lambda i, idx: (0, 0)),
            pl.BlockSpec((hidden, rating_range), lambda i, idx: (0, 0)),
            pl.BlockSpec((1, rating_range), lambda i, idx: (0, 0)),
        ],
        out_specs=pl.BlockSpec((num_users, rating_range),
                               lambda i, idx: (0, 0)),
    )
    return pl.pallas_call(
        kern,
        out_shape=jax.ShapeDtypeStruct((num_users, rating_range),
                                       jnp.float32),
        grid_spec=grid_spec,
        compiler_params=pltpu.CompilerParams(
            dimension_semantics=("arbitrary",)),
    )(item_idx, word_embeddings, w_ih, w_hh, b_ih, b_hh,
      item_table.reshape(item_table.shape[0], 1, emb_dim),
      user_emb, w_out, b_out)
